# baseline (speedup 1.0000x reference)
import functools

import jax
import jax.numpy as jnp
from jax import lax
from jax.experimental import pallas as pl
from jax.experimental.pallas import tpu as pltpu
from jax.experimental.pallas import tpu_sc as plsc

_NUM_CORES = 2
_NUM_SUBCORES = 16
_NUM_WORKERS = _NUM_CORES * _NUM_SUBCORES
_LANES = 16


def _mesh():
    return plsc.VectorSubcoreMesh(
        core_axis_name="c",
        subcore_axis_name="s",
        num_cores=_NUM_CORES,
        num_subcores=_NUM_SUBCORES,
    )


@functools.cache
def _make_detile(num_rows, dim):
    full_blocks = num_rows // 128
    tail = num_rows % 128
    per_w = full_blocks // _NUM_WORKERS
    extra = full_blocks % _NUM_WORKERS
    block_elems = 128 * dim

    @functools.partial(
        pl.kernel,
        mesh=_mesh(),
        out_type=jax.ShapeDtypeStruct((num_rows * dim,), jnp.float32),
        scratch_types=[
            pltpu.VMEM((dim, 128), jnp.float32),
            pltpu.VMEM((block_elems,), jnp.float32),
            pltpu.VMEM((max(tail, 1) * dim,), jnp.float32),
        ],
        compiler_params=pltpu.CompilerParams(
            use_tc_tiling_on_sc=True, needs_layout_passes=False
        ),
    )
    def detile(tab_t, tail1d, lin, inbuf, staging, tailv):
        wid = lax.axis_index("s") * _NUM_CORES + lax.axis_index("c")
        n_blocks = jnp.where(wid < extra, per_w + 1, per_w)
        start = wid * per_w + jnp.minimum(wid, extra)
        lanes = lax.iota(jnp.int32, _LANES)

        def body(i, carry):
            tc = start + i
            pltpu.sync_copy(tab_t.at[:, pl.ds(tc * 128, 128)], inbuf)
            for j in range(dim):
                for cc in range(128 // _LANES):
                    x = inbuf[j, pl.ds(cc * _LANES, _LANES)]
                    plsc.store_scatter(
                        staging, [lanes * dim + (cc * _LANES * dim + j)], x
                    )
            pltpu.sync_copy(staging, lin.at[pl.ds(tc * block_elems, block_elems)])
            return carry

        lax.fori_loop(0, n_blocks, body, 0)
        if tail:
            @pl.when(wid == _NUM_WORKERS - 1)
            def _():
                pltpu.sync_copy(tail1d, tailv)
                pltpu.sync_copy(
                    tailv, lin.at[pl.ds(full_blocks * block_elems, tail * dim)]
                )

    return detile


@functools.cache
def _make_gather(num_rows, dim, rows):
    assert rows % (8 * _NUM_WORKERS) == 0
    r_per_w = rows // _NUM_WORKERS

    @functools.partial(
        pl.kernel,
        mesh=_mesh(),
        out_type=jax.ShapeDtypeStruct((rows, dim), jnp.float32),
        scratch_types=[
            pltpu.VMEM((r_per_w,), jnp.int32),
            pltpu.VMEM((r_per_w, dim), jnp.float32),
            pltpu.SemaphoreType.DMA,
        ],
        compiler_params=pltpu.CompilerParams(use_tc_tiling_on_sc=False),
    )
    def gather(idx_hbm, table_hbm, out_hbm, idx_v, rows_v, sem):
        wid = lax.axis_index("s") * _NUM_CORES + lax.axis_index("c")
        base = wid * r_per_w
        pltpu.sync_copy(idx_hbm.at[pl.ds(base, r_per_w)], idx_v)
        pltpu.async_copy(table_hbm.at[idx_v], rows_v, sem).wait()
        pltpu.sync_copy(rows_v, out_hbm.at[pl.ds(base, r_per_w)])

    return gather


def _kernel_impl(inputs, table):
    batch, n_fields = inputs.shape
    num_rows, dim = table.shape
    idx = inputs.reshape(-1).astype(jnp.int32)
    tail_start = (num_rows // 128) * 128
    tail1d = lax.slice(table, (tail_start, 0), (num_rows, dim)).reshape(-1)
    lin = _make_detile(num_rows, dim)(table.T, tail1d)
    out = _make_gather(num_rows, dim, batch * n_fields)(
        idx, lin.reshape(num_rows, dim)
    )
    return out.reshape(batch, n_fields, dim)


kernel = jax.jit(_kernel_impl)


# speedup vs baseline: 1.3528x; 1.3528x over previous
import functools

import jax
import jax.numpy as jnp
from jax import lax
from jax.experimental import pallas as pl
from jax.experimental.pallas import tpu as pltpu
from jax.experimental.pallas import tpu_sc as plsc

_NUM_CORES = 2
_NUM_SUBCORES = 16
_NUM_WORKERS = _NUM_CORES * _NUM_SUBCORES
_LANES = 16


def _mesh():
    return plsc.VectorSubcoreMesh(
        core_axis_name="c",
        subcore_axis_name="s",
        num_cores=_NUM_CORES,
        num_subcores=_NUM_SUBCORES,
    )


_SB = 512


@functools.cache
def _make_detile(num_rows, dim):
    full_blocks = num_rows // _SB
    tail = num_rows % _SB
    per_w = full_blocks // _NUM_WORKERS
    extra = full_blocks % _NUM_WORKERS
    block_elems = _SB * dim

    @functools.partial(
        pl.kernel,
        mesh=_mesh(),
        out_type=jax.ShapeDtypeStruct((num_rows * dim,), jnp.float32),
        scratch_types=[
            pltpu.VMEM((dim, _SB), jnp.float32),
            pltpu.VMEM((dim, _SB), jnp.float32),
            pltpu.VMEM((block_elems,), jnp.float32),
            pltpu.VMEM((block_elems,), jnp.float32),
            pltpu.VMEM((max(tail, 1) * dim,), jnp.float32),
            pltpu.SemaphoreType.DMA,
            pltpu.SemaphoreType.DMA,
            pltpu.SemaphoreType.DMA,
            pltpu.SemaphoreType.DMA,
        ],
        compiler_params=pltpu.CompilerParams(
            use_tc_tiling_on_sc=True, needs_layout_passes=False
        ),
    )
    def detile(tab_t, tail1d, lin, in0, in1, st0, st1, tailv,
               sin0, sin1, sout0, sout1):
        wid = lax.axis_index("s") * _NUM_CORES + lax.axis_index("c")
        n_blocks = jnp.where(wid < extra, per_w + 1, per_w)
        start = wid * per_w + jnp.minimum(wid, extra)
        lanes = lax.iota(jnp.int32, _LANES)
        bufs = ((in0, st0, sin0, sout0), (in1, st1, sin1, sout1))

        def src(i):
            return tab_t.at[:, pl.ds((start + i) * _SB, _SB)]

        def dst(i):
            return lin.at[pl.ds((start + i) * block_elems, block_elems)]

        pltpu.async_copy(src(0), in0, sin0)

        @pl.when(n_blocks > 1)
        def _():
            pltpu.async_copy(src(1), in1, sin1)

        def step(i, p):
            inb, st, sin, sout = bufs[p]

            @pl.when(i >= 2)
            def _():
                pltpu.make_async_copy(st, dst(i - 2), sout).wait()

            pltpu.make_async_copy(src(i), inb, sin).wait()

            def inner(cc, carry):
                coff = cc * _LANES
                for j in range(dim):
                    x = inb[j, pl.ds(coff, _LANES)]
                    plsc.store_scatter(st, [lanes * dim + (coff * dim + j)], x)
                return carry

            lax.fori_loop(0, _SB // _LANES, inner, 0)
            pltpu.async_copy(st, dst(i), sout)

            @pl.when(i + 2 < n_blocks)
            def _():
                pltpu.async_copy(src(i + 2), inb, sin)

        def body(k, carry):
            step(2 * k, 0)

            @pl.when(2 * k + 1 < n_blocks)
            def _():
                step(2 * k + 1, 1)

            return carry

        lax.fori_loop(0, (n_blocks + 1) // 2, body, 0)

        @pl.when(n_blocks >= 2)
        def _():
            pltpu.make_async_copy(st0, dst(n_blocks - 2), sout0).wait()
            pltpu.make_async_copy(st1, dst(n_blocks - 2), sout1).wait()

        @pl.when(n_blocks == 1)
        def _():
            pltpu.make_async_copy(st0, dst(0), sout0).wait()

        if tail:
            @pl.when(wid == _NUM_WORKERS - 1)
            def _():
                pltpu.sync_copy(tail1d, tailv)
                pltpu.sync_copy(
                    tailv, lin.at[pl.ds(full_blocks * block_elems, tail * dim)]
                )

    return detile


@functools.cache
def _make_gather(num_rows, dim, rows):
    assert rows % (8 * _NUM_WORKERS) == 0
    r_per_w = rows // _NUM_WORKERS

    @functools.partial(
        pl.kernel,
        mesh=_mesh(),
        out_type=jax.ShapeDtypeStruct((rows, dim), jnp.float32),
        scratch_types=[
            pltpu.VMEM((r_per_w,), jnp.int32),
            pltpu.VMEM((r_per_w, dim), jnp.float32),
            pltpu.SemaphoreType.DMA,
        ],
        compiler_params=pltpu.CompilerParams(use_tc_tiling_on_sc=False),
    )
    def gather(idx_hbm, table_hbm, out_hbm, idx_v, rows_v, sem):
        wid = lax.axis_index("s") * _NUM_CORES + lax.axis_index("c")
        base = wid * r_per_w
        pltpu.sync_copy(idx_hbm.at[pl.ds(base, r_per_w)], idx_v)
        pltpu.async_copy(table_hbm.at[idx_v], rows_v, sem).wait()
        pltpu.sync_copy(rows_v, out_hbm.at[pl.ds(base, r_per_w)])

    return gather


def _kernel_impl(inputs, table):
    batch, n_fields = inputs.shape
    num_rows, dim = table.shape
    idx = inputs.reshape(-1).astype(jnp.int32)
    tail_start = (num_rows // _SB) * _SB
    tail1d = lax.slice(table, (tail_start, 0), (num_rows, dim)).reshape(-1)
    lin = _make_detile(num_rows, dim)(table.T, tail1d)
    out = _make_gather(num_rows, dim, batch * n_fields)(
        idx, lin.reshape(num_rows, dim)
    )
    return out.reshape(batch, n_fields, dim)


kernel = jax.jit(_kernel_impl)
